# baseline (speedup 1.0000x reference)
import functools

import jax
import jax.numpy as jnp
from jax.experimental import pallas as pl
from jax.experimental.pallas import tpu as pltpu

B = 32
N = 20000
C = 25
AB = 10000
NCHUNK = N // AB
CLAMP = 16.118095


def _loss_kernel(pred_ref, gt_ref, out_all, out_cls, out_loc, acc, eneg):
    b = pl.program_id(0)
    chunk = pl.program_id(1)

    x = pred_ref[0]
    g = gt_ref[0]

    xc = jnp.clip(x, -CLAMP, CLAMP)
    e2 = jnp.exp2(jnp.abs(xc) * (-1.4426950408889634))
    sp = jnp.maximum(xc, 0.0) + jnp.log2(1.0 + e2) * 0.6931471805599453
    bce = sp - g * xc
    l1 = jnp.abs(x - g)

    negb = g[:, 4:5]
    posb = 1.0 - negb

    ci = jax.lax.broadcasted_iota(jnp.int32, (1, C), 1)
    m_cls = jnp.where(ci >= 4, 1.0 / 21.0, 0.0).astype(jnp.float32)
    m_loc = jnp.where(ci < 4, 0.25, 0.0).astype(jnp.float32)

    w3 = jnp.concatenate([posb, negb], axis=1)
    tdot = functools.partial(
        jax.lax.dot_general,
        dimension_numbers=(((0,), (0,)), ((), ())),
        preferred_element_type=jnp.float32,
    )
    sums_b = tdot(w3, bce)
    sums_l = tdot(posb, l1)

    s_fg = jnp.sum(sums_b[0:1] * m_cls)
    s_bg = jnp.sum(sums_b[1:2] * m_cls)
    s_lc = jnp.sum(sums_l * m_loc)
    s_pos = jnp.sum(posb)

    cols_b = jax.lax.dot_general(
        bce, m_cls.reshape(C, 1),
        dimension_numbers=(((1,), (0,)), ((), ())),
        preferred_element_type=jnp.float32,
    )
    e_neg = cols_b * negb
    eneg[pl.ds(chunk * AB, AB), :] = e_neg

    @pl.when(chunk == 0)
    def _():
        acc[0] = s_pos
        acc[1] = s_fg
        acc[2] = s_bg
        acc[3] = s_lc

    @pl.when(chunk > 0)
    def _():
        acc[0] += s_pos
        acc[1] += s_fg
        acc[2] += s_bg
        acc[3] += s_lc

    @pl.when(chunk == NCHUNK - 1)
    def _():
        npos = acc[0]
        thres = npos * 3.0

        acc[7] = acc[2]

        @pl.when(thres < float(N))
        def _():
            kf = jnp.minimum(jnp.ceil(thres), float(N))
            v = eneg[:, :]
            bits = jax.lax.bitcast_convert_type(v, jnp.int32)

            def body(i, prefix):
                cand = prefix | (1 << (30 - i))
                cnt = jnp.sum((bits >= cand).astype(jnp.float32))
                return jnp.where(cnt >= kf, cand, prefix)

            t_bits = jax.lax.fori_loop(0, 31, body, jnp.int32(0))
            t = jax.lax.bitcast_convert_type(t_bits, jnp.float32)
            above = v > t
            cnt_gt = jnp.sum(above.astype(jnp.float32))
            sum_gt = jnp.sum(jnp.where(above, v, 0.0))
            acc[7] = sum_gt + (kf - cnt_gt) * t

        loss_cls_b = acc[1] + acc[7]
        loss_loc_b = acc[3]
        c_all = (loss_cls_b + loss_loc_b) / npos
        c_cls = loss_cls_b / npos
        c_loc = loss_loc_b / npos

        @pl.when(b == 0)
        def _():
            acc[4] = c_all
            acc[5] = c_cls
            acc[6] = c_loc

        @pl.when(b > 0)
        def _():
            acc[4] += c_all
            acc[5] += c_cls
            acc[6] += c_loc

        @pl.when(b == B - 1)
        def _():
            out_all[0, 0] = acc[4] * (1.0 / B)
            out_cls[0, 0] = acc[5] * (1.0 / B)
            out_loc[0, 0] = acc[6] * (1.0 / B)


@jax.jit
def kernel(pred, gt):
    out_shape = [jax.ShapeDtypeStruct((1, 1), jnp.float32)] * 3
    in_spec = pl.BlockSpec((1, AB, C), lambda b, c: (b, c, 0))
    out_spec = pl.BlockSpec((1, 1), lambda b, c: (0, 0), memory_space=pltpu.SMEM)
    outs = pl.pallas_call(
        _loss_kernel,
        grid=(B, NCHUNK),
        in_specs=[in_spec, in_spec],
        out_specs=[out_spec, out_spec, out_spec],
        out_shape=out_shape,
        scratch_shapes=[
            pltpu.SMEM((8,), jnp.float32),
            pltpu.VMEM((N, 1), jnp.float32),
        ],
    )(pred, gt)
    return outs[0][0, 0], outs[1][0, 0], outs[2][0, 0]


# speedup vs baseline: 1.0714x; 1.0714x over previous
import functools

import jax
import jax.numpy as jnp
from jax.experimental import pallas as pl
from jax.experimental.pallas import tpu as pltpu

B = 32
N = 20000
C = 25
AB = 10000
NCHUNK = N // AB
CLAMP = 16.118095


def _loss_kernel(pred_ref, gt_ref, out_all, out_cls, out_loc, acc, eneg):
    b = pl.program_id(0)
    chunk = pl.program_id(1)

    x = pred_ref[0]
    g = gt_ref[0]

    xc = jnp.clip(x, -CLAMP, CLAMP)
    e2 = jnp.exp2(jnp.abs(xc) * (-1.4426950408889634))
    sp = jnp.maximum(xc, 0.0) + jnp.log2(1.0 + e2) * 0.6931471805599453
    bce = sp - g * xc
    l1 = jnp.abs(x - g)

    negb = g[:, 4:5]
    posb = 1.0 - negb

    li = jax.lax.broadcasted_iota(jnp.int32, (C, 2), 0)
    gi = jax.lax.broadcasted_iota(jnp.int32, (C, 2), 1)
    m_right = jnp.where(
        (li >= 4) == (gi == 0),
        jnp.where(gi == 0, 1.0 / 21.0, 0.25),
        0.0,
    ).astype(jnp.float32)

    dot = functools.partial(
        jax.lax.dot_general,
        dimension_numbers=(((1,), (0,)), ((), ())),
        preferred_element_type=jnp.float32,
    )
    cols_b = dot(bce, m_right[:, 0:1])
    cols_l = dot(l1, m_right[:, 1:2])
    e_neg = cols_b * negb
    ones_col = jnp.ones((AB, 1), jnp.float32)
    lhs = jnp.concatenate([ones_col, posb], axis=1)
    rhs = jnp.concatenate([cols_b, cols_l, e_neg, ones_col], axis=1)
    sums = jax.lax.dot_general(
        lhs, rhs,
        dimension_numbers=(((0,), (0,)), ((), ())),
        preferred_element_type=jnp.float32,
    )
    s_bg = sums[0, 2]
    s_fg = sums[1, 0]
    s_lc = sums[1, 1]
    s_pos = sums[1, 3]
    eneg[pl.ds(chunk * AB, AB), :] = e_neg

    @pl.when(chunk == 0)
    def _():
        acc[0] = s_pos
        acc[1] = s_fg
        acc[2] = s_bg
        acc[3] = s_lc

    @pl.when(chunk > 0)
    def _():
        acc[0] += s_pos
        acc[1] += s_fg
        acc[2] += s_bg
        acc[3] += s_lc

    @pl.when(chunk == NCHUNK - 1)
    def _():
        npos = acc[0]
        thres = npos * 3.0

        acc[7] = acc[2]

        @pl.when(thres < float(N))
        def _():
            kf = jnp.minimum(jnp.ceil(thres), float(N))
            v = eneg[:, :]
            bits = jax.lax.bitcast_convert_type(v, jnp.int32)

            def body(i, prefix):
                cand = prefix | (1 << (30 - i))
                cnt = jnp.sum((bits >= cand).astype(jnp.float32))
                return jnp.where(cnt >= kf, cand, prefix)

            t_bits = jax.lax.fori_loop(0, 31, body, jnp.int32(0))
            t = jax.lax.bitcast_convert_type(t_bits, jnp.float32)
            above = v > t
            cnt_gt = jnp.sum(above.astype(jnp.float32))
            sum_gt = jnp.sum(jnp.where(above, v, 0.0))
            acc[7] = sum_gt + (kf - cnt_gt) * t

        loss_cls_b = acc[1] + acc[7]
        loss_loc_b = acc[3]
        c_all = (loss_cls_b + loss_loc_b) / npos
        c_cls = loss_cls_b / npos
        c_loc = loss_loc_b / npos

        @pl.when(b == 0)
        def _():
            acc[4] = c_all
            acc[5] = c_cls
            acc[6] = c_loc

        @pl.when(b > 0)
        def _():
            acc[4] += c_all
            acc[5] += c_cls
            acc[6] += c_loc

        @pl.when(b == B - 1)
        def _():
            out_all[0, 0] = acc[4] * (1.0 / B)
            out_cls[0, 0] = acc[5] * (1.0 / B)
            out_loc[0, 0] = acc[6] * (1.0 / B)


@jax.jit
def kernel(pred, gt):
    out_shape = [jax.ShapeDtypeStruct((1, 1), jnp.float32)] * 3
    in_spec = pl.BlockSpec((1, AB, C), lambda b, c: (b, c, 0))
    out_spec = pl.BlockSpec((1, 1), lambda b, c: (0, 0), memory_space=pltpu.SMEM)
    outs = pl.pallas_call(
        _loss_kernel,
        grid=(B, NCHUNK),
        in_specs=[in_spec, in_spec],
        out_specs=[out_spec, out_spec, out_spec],
        out_shape=out_shape,
        scratch_shapes=[
            pltpu.SMEM((8,), jnp.float32),
            pltpu.VMEM((N, 1), jnp.float32),
        ],
    )(pred, gt)
    return outs[0][0, 0], outs[1][0, 0], outs[2][0, 0]


# speedup vs baseline: 1.1737x; 1.0955x over previous
import functools

import jax
import jax.numpy as jnp
from jax.experimental import pallas as pl
from jax.experimental.pallas import tpu as pltpu

B = 32
N = 20000
C = 25
AB = 10000
NCHUNK = N // AB
CLAMP = 16.118095


def _loss_kernel(pred_ref, gt_ref, out_all, out_cls, out_loc, acc, eneg):
    b = pl.program_id(0)
    chunk = pl.program_id(1)

    x = pred_ref[0]
    g = gt_ref[0]

    xc = jnp.clip(x, -CLAMP, CLAMP)
    e2 = jnp.exp2(jnp.abs(xc) * (-1.4426950408889634))
    sp = jnp.maximum(xc, 0.0) + jnp.log2(1.0 + e2) * 0.6931471805599453
    bce = sp - g * xc
    l1 = jnp.abs(x - g)

    negb = g[:, 4:5]
    posb = 1.0 - negb

    li = jax.lax.broadcasted_iota(jnp.int32, (C, 2), 0)
    gi = jax.lax.broadcasted_iota(jnp.int32, (C, 2), 1)
    m_right = jnp.where(
        (li >= 4) == (gi == 0),
        jnp.where(gi == 0, 1.0 / 21.0, 0.25),
        0.0,
    ).astype(jnp.float32)

    dot = functools.partial(
        jax.lax.dot_general,
        dimension_numbers=(((1,), (0,)), ((), ())),
        preferred_element_type=jnp.float32,
    )
    cols_b = dot(bce, m_right[:, 0:1])
    cols_l = dot(l1, m_right[:, 1:2])
    e_neg = cols_b * negb
    ones_col = jnp.ones((AB, 1), jnp.float32)
    lhs = jnp.concatenate([ones_col, posb], axis=1)
    rhs = jnp.concatenate([cols_b, cols_l, ones_col], axis=1)
    sums = jax.lax.dot_general(
        lhs, rhs,
        dimension_numbers=(((0,), (0,)), ((), ())),
        preferred_element_type=jnp.float32,
    )
    s_ent = sums[0, 0]
    s_fg = sums[1, 0]
    s_lc = sums[1, 1]
    s_pos = sums[1, 2]
    s_bg = s_ent - s_fg
    eneg[pl.ds(chunk * AB, AB), :] = e_neg

    @pl.when(chunk == 0)
    def _():
        acc[0] = s_pos
        acc[1] = s_fg
        acc[2] = s_bg
        acc[3] = s_lc

    @pl.when(chunk > 0)
    def _():
        acc[0] += s_pos
        acc[1] += s_fg
        acc[2] += s_bg
        acc[3] += s_lc

    @pl.when(chunk == NCHUNK - 1)
    def _():
        npos = acc[0]
        thres = npos * 3.0

        acc[7] = acc[2]

        @pl.when(thres < float(N))
        def _():
            kf = jnp.minimum(jnp.ceil(thres), float(N))
            v = eneg[:, :]
            bits = jax.lax.bitcast_convert_type(v, jnp.int32)

            def body(i, prefix):
                cand = prefix | (1 << (30 - i))
                cnt = jnp.sum((bits >= cand).astype(jnp.float32))
                return jnp.where(cnt >= kf, cand, prefix)

            t_bits = jax.lax.fori_loop(0, 31, body, jnp.int32(0))
            t = jax.lax.bitcast_convert_type(t_bits, jnp.float32)
            above = v > t
            cnt_gt = jnp.sum(above.astype(jnp.float32))
            sum_gt = jnp.sum(jnp.where(above, v, 0.0))
            acc[7] = sum_gt + (kf - cnt_gt) * t

        loss_cls_b = acc[1] + acc[7]
        loss_loc_b = acc[3]
        c_all = (loss_cls_b + loss_loc_b) / npos
        c_cls = loss_cls_b / npos
        c_loc = loss_loc_b / npos

        @pl.when(b == 0)
        def _():
            acc[4] = c_all
            acc[5] = c_cls
            acc[6] = c_loc

        @pl.when(b > 0)
        def _():
            acc[4] += c_all
            acc[5] += c_cls
            acc[6] += c_loc

        @pl.when(b == B - 1)
        def _():
            out_all[0, 0] = acc[4] * (1.0 / B)
            out_cls[0, 0] = acc[5] * (1.0 / B)
            out_loc[0, 0] = acc[6] * (1.0 / B)


@jax.jit
def kernel(pred, gt):
    out_shape = [jax.ShapeDtypeStruct((1, 1), jnp.float32)] * 3
    in_spec = pl.BlockSpec((1, AB, C), lambda b, c: (b, c, 0))
    out_spec = pl.BlockSpec((1, 1), lambda b, c: (0, 0), memory_space=pltpu.SMEM)
    outs = pl.pallas_call(
        _loss_kernel,
        grid=(B, NCHUNK),
        in_specs=[in_spec, in_spec],
        out_specs=[out_spec, out_spec, out_spec],
        out_shape=out_shape,
        scratch_shapes=[
            pltpu.SMEM((8,), jnp.float32),
            pltpu.VMEM((N, 1), jnp.float32),
        ],
    )(pred, gt)
    return outs[0][0, 0], outs[1][0, 0], outs[2][0, 0]
